# baseline (speedup 1.0000x reference)
import functools

import jax
import jax.numpy as jnp
from jax import lax
from jax.experimental import pallas as pl
from jax.experimental.pallas import tpu as pltpu
from jax.experimental.pallas import tpu_sc as plsc

N, P2, TOPK, W2, CKV = 8, 49, 4, 64, 384
NW = 32
ITEMS = N * P2 * TOPK
IPW = ITEMS // NW
LANES = 16
CHUNKS = CKV // LANES
NIN = 2
NOUT = 2
GROUPS = (IPW - 1) // NOUT


def _body(ctrl_hbm, pfr_hbm, dst_hbm, w_hbm, kv_hbm, out_hbm,
          ctrl_v, pfr_v, dst_v, w_v, ibuf, obuf, in_sems, out_sems):
    nc = 2
    wid = lax.axis_index("s") * nc + lax.axis_index("c")

    pltpu.sync_copy(ctrl_hbm.at[wid], ctrl_v)
    pltpu.sync_copy(pfr_hbm.at[wid], pfr_v)
    pltpu.sync_copy(dst_hbm.at[wid], dst_v)
    pltpu.sync_copy(w_hbm.at[wid], w_v)

    def start_gather(region, slot):
        pltpu.async_copy(kv_hbm.at[region], ibuf.at[slot], in_sems.at[slot])

    def wait_in(slot):
        pltpu.make_async_copy(kv_hbm.at[0], ibuf.at[slot],
                              in_sems.at[slot]).wait()

    def start_out(dest, slot):
        pltpu.async_copy(obuf.at[slot], out_hbm.at[dest], out_sems.at[slot])

    def wait_out(slot):
        pltpu.make_async_copy(obuf.at[0], out_hbm.at[0],
                              out_sems.at[slot]).wait()

    def scale(islot, oslot, wv):
        def row(r, c2):
            for cc in range(CHUNKS):
                sl = pl.ds(cc * LANES, LANES)
                obuf[oslot, r, sl] = ibuf[islot, r, sl] * wv
            return c2

        lax.fori_loop(0, W2, row, 0)

    def item(j, oslot):
        cv = jnp.max(ctrl_v[j])
        par = (cv >> 1) & 1
        wv = w_v[j]

        @pl.when((cv & 1) == 1)
        def _():
            wait_in(par)

        @pl.when(j >= NOUT)
        def _():
            wait_out(oslot)

        scale(par, oslot, wv)

        start_out(jnp.max(dst_v[j]) & 0xFFFF, oslot)

        @pl.when(((cv >> 2) & 1) == 1)
        def _():
            start_gather(jnp.max(pfr_v[j]), par)

    c0 = jnp.max(ctrl_v[0])
    start_gather(jnp.max(pfr_v[IPW - 1]), 0)

    @pl.when(((c0 >> 3) & 1) == 1)
    def _():
        start_gather(jnp.max(dst_v[IPW - 1]) >> 16, 1)

    def group(g, carry):
        for b in range(NOUT):
            item(g * NOUT + b, b)
        return carry

    lax.fori_loop(0, GROUPS, group, 0)

    item(IPW - 1, 0)

    wait_out(1)
    wait_out(0)


@functools.partial(
    pl.kernel,
    mesh=plsc.VectorSubcoreMesh(core_axis_name="c", subcore_axis_name="s"),
    out_type=jax.ShapeDtypeStruct((ITEMS, W2, CKV), jnp.float32),
    scratch_types=[
        pltpu.VMEM((IPW, LANES), jnp.int32),
        pltpu.VMEM((IPW, LANES), jnp.int32),
        pltpu.VMEM((IPW, LANES), jnp.int32),
        pltpu.VMEM((IPW, LANES), jnp.float32),
        pltpu.VMEM((NIN, W2, CKV), jnp.float32),
        pltpu.VMEM((NOUT, W2, CKV), jnp.float32),
        pltpu.SemaphoreType.DMA((NIN,)),
        pltpu.SemaphoreType.DMA((NOUT,)),
    ],
    compiler_params=pltpu.CompilerParams(needs_layout_passes=False),
)
def _gather_scale(ctrl_hbm, pfr_hbm, dst_hbm, w_hbm, kv_hbm, out_hbm,
                  ctrl_v, pfr_v, dst_v, w_v, ibuf, obuf, in_sems, out_sems):
    _body(ctrl_hbm, pfr_hbm, dst_hbm, w_hbm, kv_hbm, out_hbm,
          ctrl_v, pfr_v, dst_v, w_v, ibuf, obuf, in_sems, out_sems)


def _bcast16(a):
    return jnp.broadcast_to(a[:, :, None], (NW, IPW, LANES))


def kernel(r_idx, r_weight, kv):
    n, p2, w2, c_kv = kv.shape
    topk = r_idx.shape[-1]
    gidx = (jnp.arange(n, dtype=jnp.int32)[:, None, None] * p2
            + r_idx.astype(jnp.int32)).reshape(-1)
    order = jnp.argsort(gidx).astype(jnp.int32)
    g = gidx[order].reshape(NW, IPW)
    wsort = r_weight.astype(jnp.float32).reshape(-1)[order].reshape(NW, IPW)
    dest = order.reshape(NW, IPW)

    prev = jnp.concatenate(
        [jnp.full((NW, 1), -1, jnp.int32), g[:, :-1]], axis=1)
    first = (g != prev).astype(jnp.int32)
    runidx = jnp.cumsum(first, axis=1) - 1
    nxt = jnp.concatenate(
        [g[:, 1:], jnp.full((NW, 1), -1, jnp.int32)], axis=1)
    last = g != nxt
    nruns = runidx[:, -1] + 1
    rows = jnp.arange(NW, dtype=jnp.int32)[:, None]
    region_of_run = jnp.zeros((NW, IPW), jnp.int32).at[
        jnp.broadcast_to(rows, (NW, IPW)), runidx].set(g)
    pf_region = jnp.take_along_axis(
        region_of_run, jnp.minimum(runidx + 2, IPW - 1), axis=1)
    pf_valid = (last & (runidx + 2 < nruns[:, None])).astype(jnp.int32)
    has_run1 = (nruns > 1).astype(jnp.int32)

    ctrl = (first | ((runidx & 1) << 1) | (pf_valid << 2)
            | (has_run1[:, None] << 3) * (jnp.arange(IPW) == 0))
    pfr = jnp.where(jnp.arange(IPW) == IPW - 1,
                    region_of_run[:, :1], pf_region)
    dst = dest | ((region_of_run[:, 1:2] << 16)
                  * (jnp.arange(IPW) == IPW - 1))

    out = _gather_scale(_bcast16(ctrl.astype(jnp.int32)),
                        _bcast16(pfr.astype(jnp.int32)),
                        _bcast16(dst.astype(jnp.int32)),
                        _bcast16(wsort),
                        kv.reshape(n * p2, w2, c_kv))
    return out.reshape(n, p2, topk, w2, c_kv)


# speedup vs baseline: 3.1629x; 3.1629x over previous
import functools

import jax
import jax.numpy as jnp
from jax import lax
from jax.experimental import pallas as pl
from jax.experimental.pallas import tpu as pltpu
from jax.experimental.pallas import tpu_sc as plsc

N, P2, TOPK, W2, CKV = 8, 49, 4, 64, 384
NW = 32
ITEMS = N * P2 * TOPK
IPW = ITEMS // NW
LANES = 16
CHUNKS = CKV // LANES
NBUF = 4
GROUPS = (IPW - 1) // NBUF


def _body(bidx_hbm, w_hbm, kv_hbm, out_hbm, bidx_v, w_v, buf,
          in_sems, out_sems):
    nc = 2
    wid = lax.axis_index("s") * nc + lax.axis_index("c")

    pltpu.sync_copy(bidx_hbm.at[wid], bidx_v)
    pltpu.sync_copy(w_hbm.at[wid], w_v)

    def start_gather(item, slot):
        base = jnp.max(bidx_v[item])
        pltpu.async_copy(kv_hbm.at[base], buf.at[slot], in_sems.at[slot])

    def wait_in(slot):
        pltpu.make_async_copy(kv_hbm.at[0], buf.at[slot],
                              in_sems.at[slot]).wait()

    def start_out(item, slot):
        pltpu.async_copy(buf.at[slot], out_hbm.at[wid * IPW + item],
                         out_sems.at[slot])

    def wait_out(slot):
        pltpu.make_async_copy(buf.at[slot], out_hbm.at[0],
                              out_sems.at[slot]).wait()

    def scale(item, slot):
        wv = w_v[item]

        def row(r, c2):
            for cc in range(CHUNKS):
                sl = pl.ds(cc * LANES, LANES)
                buf[slot, r, sl] = buf[slot, r, sl] * wv
            return c2

        lax.fori_loop(0, W2, row, 0)

    start_gather(0, 0)
    start_gather(1, 1)

    def group(g, carry):
        for b in range(NBUF):
            j = g * NBUF + b
            b2 = (b + 2) % NBUF
            @pl.when(j >= 2)
            def _():
                wait_out(b2)

            @pl.when(j + 2 < IPW)
            def _():
                start_gather(j + 2, b2)

            wait_in(b)
            scale(j, b)
            start_out(j, b)
        return carry

    lax.fori_loop(0, GROUPS, group, 0)

    last = IPW - 1
    wait_in(last % NBUF)
    scale(last, last % NBUF)
    start_out(last, last % NBUF)

    wait_out(2)
    wait_out(3)
    wait_out(0)


@functools.partial(
    pl.kernel,
    mesh=plsc.VectorSubcoreMesh(core_axis_name="c", subcore_axis_name="s"),
    out_type=jax.ShapeDtypeStruct((ITEMS, W2, CKV), jnp.float32),
    scratch_types=[
        pltpu.VMEM((IPW, LANES), jnp.int32),
        pltpu.VMEM((IPW, LANES), jnp.float32),
        pltpu.VMEM((NBUF, W2, CKV), jnp.float32),
        pltpu.SemaphoreType.DMA((NBUF,)),
        pltpu.SemaphoreType.DMA((NBUF,)),
    ],
    compiler_params=pltpu.CompilerParams(needs_layout_passes=False),
)
def _gather_scale(bidx_hbm, w_hbm, kv_hbm, out_hbm, bidx_v, w_v, buf,
                  in_sems, out_sems):
    _body(bidx_hbm, w_hbm, kv_hbm, out_hbm, bidx_v, w_v, buf,
          in_sems, out_sems)


def kernel(r_idx, r_weight, kv):
    n, p2, w2, c_kv = kv.shape
    topk = r_idx.shape[-1]
    base = (jnp.arange(n, dtype=jnp.int32)[:, None, None] * p2
            + r_idx.astype(jnp.int32)).reshape(NW, IPW)
    bidx = jnp.broadcast_to(base[:, :, None], (NW, IPW, LANES))
    wgt = jnp.broadcast_to(
        r_weight.astype(jnp.float32).reshape(NW, IPW)[:, :, None],
        (NW, IPW, LANES))
    kvr = kv.reshape(n * p2, w2, c_kv)
    out = _gather_scale(bidx, wgt, kvr)
    return out.reshape(n, p2, topk, w2, c_kv)
